# baseline (speedup 1.0000x reference)
import functools

import jax
import jax.numpy as jnp
from jax import lax
from jax.experimental import pallas as pl
from jax.experimental.pallas import tpu as pltpu
from jax.experimental.pallas import tpu_sc as plsc

_NUM_CORES = 2
_NUM_SUBCORES = 16
_NUM_WORKERS = _NUM_CORES * _NUM_SUBCORES
_LANES = 16


def _sc_pool_fn(batch, ctx, vocab, embed):
  bpw = batch // _NUM_WORKERS
  lanes_per_row = embed // _LANES
  inv_ctx = 1.0 / ctx

  def body(table_hbm, ids_hbm, out_hbm, idx_v, rows_a, rows_b, acc_v,
           sem_a, sem_b, sem_0):
    wid = lax.axis_index("s") * _NUM_CORES + lax.axis_index("c")
    base = wid * bpw
    pltpu.sync_copy(ids_hbm.at[:, pl.ds(base, bpw)], idx_v)

    bufs = (rows_a, rows_b)
    sems = (sem_a, sem_b)

    cp0 = pltpu.async_copy(table_hbm.at[idx_v.at[0]], acc_v, sem_0)
    inflight = pltpu.async_copy(table_hbm.at[idx_v.at[1]], bufs[1], sems[1])
    cp0.wait()

    for j in range(1, ctx):
      nxt = None
      if j + 1 < ctx:
        nxt = pltpu.async_copy(
            table_hbm.at[idx_v.at[j + 1]], bufs[(j + 1) % 2],
            sems[(j + 1) % 2])
      inflight.wait()
      buf = bufs[j % 2]

      def add_row(r, carry, buf=buf):
        for c in range(lanes_per_row):
          sl = pl.ds(c * _LANES, _LANES)
          acc_v[r, sl] = acc_v[r, sl] + buf[r, sl]
        return carry

      lax.fori_loop(0, bpw, add_row, 0)
      inflight = nxt

    def scale_row(r, carry):
      for c in range(lanes_per_row):
        sl = pl.ds(c * _LANES, _LANES)
        acc_v[r, sl] = acc_v[r, sl] * inv_ctx
      return carry

    lax.fori_loop(0, bpw, scale_row, 0)
    pltpu.sync_copy(acc_v, out_hbm.at[pl.ds(base, bpw)])

  return pl.kernel(
      body,
      out_type=jax.ShapeDtypeStruct((batch, embed), jnp.float32),
      mesh=plsc.VectorSubcoreMesh(core_axis_name="c", subcore_axis_name="s"),
      compiler_params=pltpu.CompilerParams(use_tc_tiling_on_sc=False),
      scratch_types=[
          pltpu.VMEM((ctx, bpw), jnp.int32),
          pltpu.VMEM((bpw, embed), jnp.float32),
          pltpu.VMEM((bpw, embed), jnp.float32),
          pltpu.VMEM((bpw, embed), jnp.float32),
          pltpu.SemaphoreType.DMA,
          pltpu.SemaphoreType.DMA,
          pltpu.SemaphoreType.DMA,
      ],
  )


def _mm_body(p_ref, w_ref, b_ref, o_ref):
  o_ref[...] = lax.dot_general(
      p_ref[...], w_ref[...],
      dimension_numbers=(((1,), (1,)), ((), ())),
      preferred_element_type=jnp.float32) + b_ref[...]


def _mm_fn(batch, vocab, embed, n_tile):
  grid = (pl.cdiv(vocab, n_tile),)
  return pl.pallas_call(
      _mm_body,
      grid=grid,
      in_specs=[
          pl.BlockSpec((batch, embed), lambda i: (0, 0)),
          pl.BlockSpec((n_tile, embed), lambda i: (i, 0)),
          pl.BlockSpec((1, n_tile), lambda i: (0, i)),
      ],
      out_specs=pl.BlockSpec((batch, n_tile), lambda i: (0, i)),
      out_shape=jax.ShapeDtypeStruct((batch, vocab), jnp.float32),
  )


@functools.lru_cache(maxsize=None)
def _build(batch, ctx, vocab, embed):
  return _sc_pool_fn(batch, ctx, vocab, embed), _mm_fn(batch, vocab, embed, 512)


def kernel(context_ids, embed_table, fc_weight, fc_bias):
  batch, ctx = context_ids.shape
  vocab, embed = embed_table.shape
  sc_pool, mm = _build(batch, ctx, vocab, embed)
  ids_t = context_ids.astype(jnp.int32).T
  pooled = sc_pool(embed_table, ids_t)
  return mm(pooled, fc_weight, fc_bias.reshape(1, vocab))
